# baseline (speedup 1.0000x reference)
import functools

import jax
import jax.numpy as jnp
from jax import lax
from jax.experimental import pallas as pl
from jax.experimental.pallas import tpu as pltpu
from jax.experimental.pallas import tpu_sc as plsc

D = 128
CHUNK = 128
NBUF = 5
EPS = 1e-5
BB = 8


def _make_sc_gather(B, S, V):
    info = plsc.get_sparse_core_info()
    NC, NS = info.num_cores, info.num_subcores
    NW = NC * NS
    N = B * S
    assert N % (NW * CHUNK) == 0
    rows_per_w = N // NW
    n_chunks = rows_per_w // CHUNK
    assert n_chunks % NBUF == 0

    mesh = plsc.VectorSubcoreMesh(core_axis_name="c", subcore_axis_name="s")

    scratch_types = (
        [pltpu.VMEM((n_chunks, CHUNK), jnp.int32)]
        + [pltpu.VMEM((CHUNK, D), jnp.float32) for _ in range(NBUF)]
        + [pltpu.SemaphoreType.DMA for _ in range(2 * NBUF)]
    )

    @functools.partial(
        pl.kernel,
        out_type=jax.ShapeDtypeStruct((N, D), jnp.float32),
        mesh=mesh,
        scratch_types=scratch_types,
    )
    def sc_gather(ids_hbm, word_hbm, out_hbm, idx_v, *rest):
        bufs = rest[:NBUF]
        gsems = rest[NBUF:2 * NBUF]
        osems = rest[2 * NBUF:]

        cid = lax.axis_index("c")
        sid = lax.axis_index("s")
        wid = sid * NC + cid
        out_base = wid * rows_per_w

        pltpu.sync_copy(ids_hbm.at[wid], idx_v)

        def wait_gather(b, jc):
            pltpu.make_async_copy(
                word_hbm.at[idx_v.at[jc]], bufs[b], gsems[b]).wait()

        def wait_out(b):
            pltpu.make_async_copy(
                bufs[b], out_hbm.at[pl.ds(0, CHUNK)], osems[b]).wait()

        for b in range(NBUF - 1):
            pltpu.async_copy(word_hbm.at[idx_v.at[b]], bufs[b], gsems[b])

        def outer_body(g, carry):
            for b in range(NBUF):
                jc = NBUF * g + b
                wait_gather(b, jc)
                pltpu.async_copy(
                    bufs[b],
                    out_hbm.at[pl.ds(out_base + jc * CHUNK, CHUNK)],
                    osems[b])
                pb = (b + NBUF - 1) % NBUF
                nc = jc + NBUF - 1

                @pl.when(jc > 0)
                def _():
                    wait_out(pb)

                @pl.when(nc < n_chunks)
                def _():
                    pltpu.async_copy(
                        word_hbm.at[idx_v.at[nc]], bufs[pb], gsems[pb])
            return carry

        lax.fori_loop(0, n_chunks // NBUF, outer_body, 0)
        wait_out((n_chunks - 1) % NBUF)

    return sc_gather


def _ln_body(x_ref, pos_ref, seg_ref, g_ref, b_ref, o_ref):
    bias = pos_ref[...] + seg_ref[0:1, :]
    x = x_ref[...] + bias[None, :, :]
    mean = jnp.mean(x, axis=-1, keepdims=True)
    xc = x - mean
    var = jnp.mean(xc * xc, axis=-1, keepdims=True)
    inv = lax.rsqrt(var + EPS)
    o_ref[...] = xc * inv * g_ref[...] + b_ref[...]


def kernel(input_ids, word_table, pos_table, seg_table, gamma, beta):
    B, S = input_ids.shape
    V, d = word_table.shape
    assert d == D
    N = B * S
    NW = 32
    ids3d = input_ids.astype(jnp.int32).reshape(NW, N // (NW * CHUNK), CHUNK)
    gathered = _make_sc_gather(B, S, V)(ids3d, word_table)

    assert B % BB == 0
    out = pl.pallas_call(
        _ln_body,
        grid=(B // BB,),
        in_specs=[
            pl.BlockSpec((BB, S, D), lambda i: (i, 0, 0)),
            pl.BlockSpec((S, D), lambda i: (0, 0)),
            pl.BlockSpec((2, D), lambda i: (0, 0)),
            pl.BlockSpec((D,), lambda i: (0,)),
            pl.BlockSpec((D,), lambda i: (0,)),
        ],
        out_specs=pl.BlockSpec((BB, S, D), lambda i: (i, 0, 0)),
        out_shape=jax.ShapeDtypeStruct((B, S, D), jnp.float32),
    )(gathered.reshape(B, S, D), pos_table[:S], seg_table, gamma, beta)
    return out


# speedup vs baseline: 1.2888x; 1.2888x over previous
import functools

import jax
import jax.numpy as jnp
from jax import lax
from jax.experimental import pallas as pl
from jax.experimental.pallas import tpu as pltpu
from jax.experimental.pallas import tpu_sc as plsc

D = 128
CHUNK = 128
NBUF = 5
EPS = 1e-5
BB = 32


def _make_sc_gather(B, S, V):
    info = plsc.get_sparse_core_info()
    NC, NS = info.num_cores, info.num_subcores
    NW = NC * NS
    N = B * S
    assert N % (NW * CHUNK) == 0
    rows_per_w = N // NW
    n_chunks = rows_per_w // CHUNK
    assert n_chunks % NBUF == 0

    mesh = plsc.VectorSubcoreMesh(core_axis_name="c", subcore_axis_name="s")

    scratch_types = (
        [pltpu.VMEM((n_chunks, CHUNK), jnp.int32)]
        + [pltpu.VMEM((CHUNK, D), jnp.float32) for _ in range(NBUF)]
        + [pltpu.SemaphoreType.DMA for _ in range(2 * NBUF)]
    )

    @functools.partial(
        pl.kernel,
        out_type=jax.ShapeDtypeStruct((N, D), jnp.float32),
        mesh=mesh,
        scratch_types=scratch_types,
    )
    def sc_gather(ids_hbm, word_hbm, out_hbm, idx_v, *rest):
        bufs = rest[:NBUF]
        gsems = rest[NBUF:2 * NBUF]
        osems = rest[2 * NBUF:]

        cid = lax.axis_index("c")
        sid = lax.axis_index("s")
        wid = sid * NC + cid
        out_base = wid * rows_per_w

        pltpu.sync_copy(ids_hbm.at[wid], idx_v)

        def wait_gather(b, jc):
            pltpu.make_async_copy(
                word_hbm.at[idx_v.at[jc]], bufs[b], gsems[b]).wait()

        def wait_out(b):
            pltpu.make_async_copy(
                bufs[b], out_hbm.at[pl.ds(0, CHUNK)], osems[b]).wait()

        for b in range(NBUF - 1):
            pltpu.async_copy(word_hbm.at[idx_v.at[b]], bufs[b], gsems[b])

        def outer_body(g, carry):
            for b in range(NBUF):
                jc = NBUF * g + b
                wait_gather(b, jc)
                pltpu.async_copy(
                    bufs[b],
                    out_hbm.at[pl.ds(out_base + jc * CHUNK, CHUNK)],
                    osems[b])
                pb = (b + NBUF - 1) % NBUF
                nc = jc + NBUF - 1

                @pl.when(jc > 0)
                def _():
                    wait_out(pb)

                @pl.when(nc < n_chunks)
                def _():
                    pltpu.async_copy(
                        word_hbm.at[idx_v.at[nc]], bufs[pb], gsems[pb])
            return carry

        lax.fori_loop(0, n_chunks // NBUF, outer_body, 0)
        wait_out((n_chunks - 1) % NBUF)

    return sc_gather


def _ln_body(x_ref, pos_ref, seg_ref, g_ref, b_ref, o_ref):
    bias = pos_ref[...] + seg_ref[0:1, :]
    x = x_ref[...] + bias[None, :, :]
    mean = jnp.mean(x, axis=-1, keepdims=True)
    xc = x - mean
    var = jnp.mean(xc * xc, axis=-1, keepdims=True)
    inv = lax.rsqrt(var + EPS)
    o_ref[...] = xc * inv * g_ref[...] + b_ref[...]


def kernel(input_ids, word_table, pos_table, seg_table, gamma, beta):
    B, S = input_ids.shape
    V, d = word_table.shape
    assert d == D
    N = B * S
    NW = 32
    ids3d = input_ids.astype(jnp.int32).reshape(NW, N // (NW * CHUNK), CHUNK)
    gathered = _make_sc_gather(B, S, V)(ids3d, word_table)

    assert B % BB == 0
    out = pl.pallas_call(
        _ln_body,
        grid=(B // BB,),
        in_specs=[
            pl.BlockSpec((BB, S, D), lambda i: (i, 0, 0)),
            pl.BlockSpec((S, D), lambda i: (0, 0)),
            pl.BlockSpec((2, D), lambda i: (0, 0)),
            pl.BlockSpec((D,), lambda i: (0,)),
            pl.BlockSpec((D,), lambda i: (0,)),
        ],
        out_specs=pl.BlockSpec((BB, S, D), lambda i: (i, 0, 0)),
        out_shape=jax.ShapeDtypeStruct((B, S, D), jnp.float32),
    )(gathered.reshape(B, S, D), pos_table[:S], seg_table, gamma, beta)
    return out


# speedup vs baseline: 1.3540x; 1.0505x over previous
import functools

import jax
import jax.numpy as jnp
from jax import lax
from jax.experimental import pallas as pl
from jax.experimental.pallas import tpu as pltpu
from jax.experimental.pallas import tpu_sc as plsc

D = 128
CHUNK = 128
NBUF = 5
EPS = 1e-5
BB = 64


def _make_sc_gather(B, S, V):
    info = plsc.get_sparse_core_info()
    NC, NS = info.num_cores, info.num_subcores
    NW = NC * NS
    N = B * S
    assert N % (NW * CHUNK) == 0
    rows_per_w = N // NW
    n_chunks = rows_per_w // CHUNK
    assert n_chunks % NBUF == 0

    mesh = plsc.VectorSubcoreMesh(core_axis_name="c", subcore_axis_name="s")

    scratch_types = (
        [pltpu.VMEM((n_chunks, CHUNK), jnp.int32)]
        + [pltpu.VMEM((CHUNK, D), jnp.float32) for _ in range(NBUF)]
        + [pltpu.SemaphoreType.DMA for _ in range(2 * NBUF)]
    )

    @functools.partial(
        pl.kernel,
        out_type=jax.ShapeDtypeStruct((N, D), jnp.float32),
        mesh=mesh,
        scratch_types=scratch_types,
    )
    def sc_gather(ids_hbm, word_hbm, out_hbm, idx_v, *rest):
        bufs = rest[:NBUF]
        gsems = rest[NBUF:2 * NBUF]
        osems = rest[2 * NBUF:]

        cid = lax.axis_index("c")
        sid = lax.axis_index("s")
        wid = sid * NC + cid
        out_base = wid * rows_per_w

        pltpu.sync_copy(ids_hbm.at[wid], idx_v)

        def wait_gather(b, jc):
            pltpu.make_async_copy(
                word_hbm.at[idx_v.at[jc]], bufs[b], gsems[b]).wait()

        def wait_out(b):
            pltpu.make_async_copy(
                bufs[b], out_hbm.at[pl.ds(0, CHUNK)], osems[b]).wait()

        for b in range(NBUF - 1):
            pltpu.async_copy(word_hbm.at[idx_v.at[b]], bufs[b], gsems[b])

        def outer_body(g, carry):
            for b in range(NBUF):
                jc = NBUF * g + b
                wait_gather(b, jc)
                pltpu.async_copy(
                    bufs[b],
                    out_hbm.at[pl.ds(out_base + jc * CHUNK, CHUNK)],
                    osems[b])
                pb = (b + NBUF - 1) % NBUF
                nc = jc + NBUF - 1

                @pl.when(jc > 0)
                def _():
                    wait_out(pb)

                @pl.when(nc < n_chunks)
                def _():
                    pltpu.async_copy(
                        word_hbm.at[idx_v.at[nc]], bufs[pb], gsems[pb])
            return carry

        lax.fori_loop(0, n_chunks // NBUF, outer_body, 0)
        wait_out((n_chunks - 1) % NBUF)

    return sc_gather


def _ln_body(x_ref, pos_ref, seg_ref, g_ref, b_ref, o_ref):
    bias = pos_ref[...] + seg_ref[0:1, :]
    x = x_ref[...] + bias[None, :, :]
    mean = jnp.mean(x, axis=-1, keepdims=True)
    xc = x - mean
    var = jnp.mean(xc * xc, axis=-1, keepdims=True)
    inv = lax.rsqrt(var + EPS)
    o_ref[...] = xc * inv * g_ref[...] + b_ref[...]


def kernel(input_ids, word_table, pos_table, seg_table, gamma, beta):
    B, S = input_ids.shape
    V, d = word_table.shape
    assert d == D
    N = B * S
    NW = 32
    ids3d = input_ids.astype(jnp.int32).reshape(NW, N // (NW * CHUNK), CHUNK)
    gathered = _make_sc_gather(B, S, V)(ids3d, word_table)

    assert B % BB == 0
    out = pl.pallas_call(
        _ln_body,
        grid=(B // BB,),
        in_specs=[
            pl.BlockSpec((BB, S, D), lambda i: (i, 0, 0)),
            pl.BlockSpec((S, D), lambda i: (0, 0)),
            pl.BlockSpec((2, D), lambda i: (0, 0)),
            pl.BlockSpec((D,), lambda i: (0,)),
            pl.BlockSpec((D,), lambda i: (0,)),
        ],
        out_specs=pl.BlockSpec((BB, S, D), lambda i: (i, 0, 0)),
        out_shape=jax.ShapeDtypeStruct((B, S, D), jnp.float32),
    )(gathered.reshape(B, S, D), pos_table[:S], seg_table, gamma, beta)
    return out
